# baseline (speedup 1.0000x reference)
import functools

import jax
import jax.numpy as jnp
from jax import lax
from jax.experimental import pallas as pl
from jax.experimental.pallas import tpu as pltpu
from jax.experimental.pallas import tpu_sc as plsc

_N = 10000
_NP = 10240
_NC = 2
_NS = 16
_NW = _NC * _NS
_CH = 128
_GP = 4


def _sc_mesh():
    return plsc.VectorSubcoreMesh(
        core_axis_name="c", subcore_axis_name="s",
        num_cores=_NC, num_subcores=_NS)

_SC_PARAMS = dict(
    compiler_params=pltpu.CompilerParams(
        needs_layout_passes=False, use_tc_tiling_on_sc=False))


@functools.lru_cache(maxsize=None)
def _deg_kernel(kc):
    @functools.partial(
        pl.kernel,
        out_type=jax.ShapeDtypeStruct((_NW, _NP), jnp.float32),
        mesh=_sc_mesh(),
        scratch_types=[
            pltpu.VMEM((kc, _CH), jnp.int32),
            pltpu.VMEM((_NP,), jnp.float32),
        ],
        **_SC_PARAMS,
    )
    def deg_k(dst_hbm, out_hbm, dst_v, deg_v):
        cid = lax.axis_index("c")
        sid = lax.axis_index("s")
        wid = cid * _NS + sid
        pltpu.sync_copy(dst_hbm.at[wid], dst_v)
        zeros16 = jnp.zeros((16,), jnp.float32)

        @pl.loop(0, _NP // 16)
        def _(i):
            deg_v[pl.ds(i * 16, 16)] = zeros16

        ones16 = jnp.ones((16,), jnp.float32)

        @pl.loop(0, kc)
        def _(j):
            for c in range(_CH // 16):
                idx = dst_v[j, pl.ds(c * 16, 16)]
                plsc.addupdate_scatter(deg_v, [idx], ones16)

        pltpu.sync_copy(deg_v, out_hbm.at[wid])

    return deg_k


@functools.lru_cache(maxsize=None)
def _scatter_kernel(feat, kc):
    rpt = _NP // _NS
    gr = _GP * _CH

    @functools.partial(
        pl.kernel,
        out_type=jax.ShapeDtypeStruct((_NC, _NP, feat), jnp.float32),
        mesh=_sc_mesh(),
        scratch_types=[
            pltpu.VMEM((kc, _CH), jnp.int32),
            pltpu.VMEM((kc, _CH), jnp.int32),
            pltpu.VMEM((gr, feat), jnp.float32),
            pltpu.VMEM_SHARED((_NP, feat), jnp.float32),
            pltpu.SemaphoreType.DMA,
            pltpu.SemaphoreType.DMA,
        ],
        **_SC_PARAMS,
    )
    def scat_k(tab_hbm, src_hbm, dst_hbm, out_hbm,
               src_v, dst_v, rows_v, acc, gsem, ssem):
        cid = lax.axis_index("c")
        sid = lax.axis_index("s")
        wid = cid * _NS + sid
        zeros16 = jnp.zeros((16,), jnp.float32)

        @pl.loop(0, gr)
        def _(r):
            for c in range(feat // 16):
                rows_v[r, pl.ds(c * 16, 16)] = zeros16

        base = sid * rpt
        pltpu.sync_copy(rows_v, acc.at[pl.ds(base, gr)])
        pltpu.sync_copy(rows_v.at[pl.ds(0, rpt - gr)],
                        acc.at[pl.ds(base + gr, rpt - gr)])
        pltpu.sync_copy(src_hbm.at[wid], src_v)
        pltpu.sync_copy(dst_hbm.at[wid], dst_v)
        plsc.subcore_barrier()

        @pl.loop(0, kc)
        def _(j):
            rv = rows_v.at[pl.ds(0, _CH)]
            pltpu.async_copy(tab_hbm.at[src_v.at[j]], rv, gsem).wait()
            pltpu.sync_copy(rv, acc.at[dst_v.at[j]], add=True)

        plsc.subcore_barrier()
        pltpu.sync_copy(acc.at[pl.ds(base, rpt)],
                        out_hbm.at[cid, pl.ds(base, rpt)])

    return scat_k


def _tc_layer1(x_pad, deg_col, w1):
    def body(x_ref, d_ref, w_ref, y_ref, dinv_ref):
        dinv = lax.rsqrt(d_ref[...])
        xw = jnp.dot(x_ref[...], w_ref[...],
                     preferred_element_type=jnp.float32)
        y_ref[...] = xw * dinv
        dinv_ref[...] = dinv

    return pl.pallas_call(
        body,
        out_shape=[
            jax.ShapeDtypeStruct((_NP, 32), jnp.float32),
            jax.ShapeDtypeStruct((_NP, 1), jnp.float32),
        ],
    )(x_pad, deg_col, w1)


def _tc_layer2(acc1, y1, dinv, b1_row, scale, w2):
    def body(a_ref, y1_ref, dinv_ref, b_ref, s_ref, w_ref, y2_ref):
        dinv = dinv_ref[...]
        agg = a_ref[0] + a_ref[1] + y1_ref[...]
        h = agg * dinv + b_ref[...]
        h = jnp.maximum(h, 0.0) * s_ref[...]
        y2_ref[...] = jnp.dot(h, w_ref[...],
                              preferred_element_type=jnp.float32) * dinv

    return pl.pallas_call(
        body,
        out_shape=jax.ShapeDtypeStruct((_NP, 64), jnp.float32),
    )(acc1, y1, dinv, b1_row, scale, w2)


def _tc_layer3(acc2, y2, dinv, b2_row):
    def body(a_ref, y2_ref, dinv_ref, b_ref, o_ref):
        z = (a_ref[0] + a_ref[1] + y2_ref[...]) * dinv_ref[...] + b_ref[...]
        o_ref[...] = jnp.where(z > 0, z, 0.01 * z)

    return pl.pallas_call(
        body,
        out_shape=jax.ShapeDtypeStruct((_NP, 64), jnp.float32),
    )(acc2, y2, dinv, b2_row)


def kernel(x, edge_index, W1, b1, W2, b2):
    n = x.shape[0]
    e = edge_index.shape[1]
    kc = -(-e // (_NW * _CH))
    kc = -(-kc // _GP) * _GP
    e_pad = _NW * kc * _CH

    fill = jnp.full((e_pad - e,), _N, jnp.int32)
    srcp = jnp.concatenate([edge_index[0], fill]).reshape(_NW, kc, _CH)
    dstp = jnp.concatenate([edge_index[1], fill]).reshape(_NW, kc, _CH)
    x_pad = jnp.pad(x, ((0, _NP - n), (0, 0)))
    mask = jax.random.bernoulli(jax.random.key(42), 0.5, (n, W1.shape[1]))
    scale = jnp.pad(jnp.where(mask, 2.0, 0.0).astype(jnp.float32),
                    ((0, _NP - n), (0, 0)))

    degp = _deg_kernel(kc)(dstp)
    deg_col = 1.0 + jnp.sum(degp, axis=0)[:, None]
    y1, dinv = _tc_layer1(x_pad, deg_col, W1)
    acc1 = _scatter_kernel(32, kc)(y1, srcp, dstp)
    y2 = _tc_layer2(acc1, y1, dinv, b1.reshape(1, -1), scale, W2)
    acc2 = _scatter_kernel(64, kc)(y2, srcp, dstp)
    out = _tc_layer3(acc2, y2, dinv, b2.reshape(1, -1))
    return out[:n]


# speedup vs baseline: 1.5523x; 1.5523x over previous
import functools

import jax
import jax.numpy as jnp
from jax import lax
from jax.experimental import pallas as pl
from jax.experimental.pallas import tpu as pltpu
from jax.experimental.pallas import tpu_sc as plsc

_N = 10000
_NP = 10240
_NC = 2
_NS = 16
_NW = _NC * _NS
_CH = 128
_KC0 = 67
_KC1 = 90
_KCM = max(_KC0, _KC1)


def _sc_mesh():
    return plsc.VectorSubcoreMesh(
        core_axis_name="c", subcore_axis_name="s",
        num_cores=_NC, num_subcores=_NS)

_SC_PARAMS = dict(
    compiler_params=pltpu.CompilerParams(
        needs_layout_passes=False, use_tc_tiling_on_sc=False))


@functools.lru_cache(maxsize=None)
def _deg_kernel():
    @functools.partial(
        pl.kernel,
        out_type=jax.ShapeDtypeStruct((_NW, _NP), jnp.float32),
        mesh=_sc_mesh(),
        scratch_types=[
            pltpu.VMEM((_KCM, _CH), jnp.int32),
            pltpu.VMEM((_NP,), jnp.float32),
        ],
        **_SC_PARAMS,
    )
    def deg_k(dst_hbm, out_hbm, dst_v, deg_v):
        cid = lax.axis_index("c")
        sid = lax.axis_index("s")
        wid = cid * _NS + sid
        nb = jnp.where(cid == 0, _KC0, _KC1)
        pltpu.sync_copy(dst_hbm.at[wid], dst_v)
        zeros16 = jnp.zeros((16,), jnp.float32)

        @pl.loop(0, _NP // 16)
        def _(i):
            deg_v[pl.ds(i * 16, 16)] = zeros16

        ones16 = jnp.ones((16,), jnp.float32)

        @pl.loop(0, nb)
        def _(j):
            for c in range(_CH // 16):
                idx = dst_v[j, pl.ds(c * 16, 16)]
                plsc.addupdate_scatter(deg_v, [idx], ones16)

        pltpu.sync_copy(deg_v, out_hbm.at[wid])

    return deg_k


@functools.lru_cache(maxsize=None)
def _scatter_kernel(feat):
    rpt = _NP // _NS

    @functools.partial(
        pl.kernel,
        out_type=jax.ShapeDtypeStruct((_NC, _NP, feat), jnp.float32),
        mesh=_sc_mesh(),
        scratch_types=[
            pltpu.VMEM((_KCM, _CH), jnp.int32),
            pltpu.VMEM((_KCM, _CH), jnp.int32),
            pltpu.VMEM((_CH, feat), jnp.float32),
            pltpu.VMEM((rpt, feat), jnp.float32),
            pltpu.VMEM_SHARED((_NP, feat), jnp.float32),
            pltpu.SemaphoreType.DMA,
        ],
        **_SC_PARAMS,
    )
    def scat_k(tab_hbm, src_hbm, dst_hbm, out_hbm,
               src_v, dst_v, rows_v, zbuf, acc, gsem):
        cid = lax.axis_index("c")
        sid = lax.axis_index("s")
        wid = cid * _NS + sid
        nb = jnp.where(cid == 0, _KC0, _KC1)
        zeros16 = jnp.zeros((16,), jnp.float32)

        @pl.loop(0, rpt)
        def _(r):
            for c in range(feat // 16):
                zbuf[r, pl.ds(c * 16, 16)] = zeros16

        base = sid * rpt
        pltpu.sync_copy(zbuf, acc.at[pl.ds(base, rpt)])
        pltpu.sync_copy(src_hbm.at[wid], src_v)
        pltpu.sync_copy(dst_hbm.at[wid], dst_v)
        plsc.subcore_barrier()

        @pl.loop(0, nb)
        def _(j):
            pltpu.async_copy(tab_hbm.at[src_v.at[j]], rows_v, gsem).wait()
            pltpu.sync_copy(rows_v, acc.at[dst_v.at[j]], add=True)

        plsc.subcore_barrier()
        pltpu.sync_copy(acc.at[pl.ds(base, rpt)],
                        out_hbm.at[cid, pl.ds(base, rpt)])

    return scat_k


def _tc_layer1(x_pad, deg_col, w1):
    def body(x_ref, d_ref, w_ref, y_ref, dinv_ref):
        dinv = lax.rsqrt(d_ref[...])
        xw = jnp.dot(x_ref[...], w_ref[...],
                     preferred_element_type=jnp.float32)
        y_ref[...] = xw * dinv
        dinv_ref[...] = dinv

    return pl.pallas_call(
        body,
        out_shape=[
            jax.ShapeDtypeStruct((_NP, 32), jnp.float32),
            jax.ShapeDtypeStruct((_NP, 1), jnp.float32),
        ],
    )(x_pad, deg_col, w1)


def _tc_layer2(acc1, y1, dinv, b1_row, scale, w2):
    def body(a_ref, y1_ref, dinv_ref, b_ref, s_ref, w_ref, y2_ref):
        dinv = dinv_ref[...]
        agg = a_ref[0] + a_ref[1] + y1_ref[...]
        h = agg * dinv + b_ref[...]
        h = jnp.maximum(h, 0.0) * s_ref[...]
        y2_ref[...] = jnp.dot(h, w_ref[...],
                              preferred_element_type=jnp.float32) * dinv

    return pl.pallas_call(
        body,
        out_shape=jax.ShapeDtypeStruct((_NP, 64), jnp.float32),
    )(acc1, y1, dinv, b1_row, scale, w2)


def _tc_layer3(acc2, y2, dinv, b2_row):
    def body(a_ref, y2_ref, dinv_ref, b_ref, o_ref):
        z = (a_ref[0] + a_ref[1] + y2_ref[...]) * dinv_ref[...] + b_ref[...]
        o_ref[...] = jnp.where(z > 0, z, 0.01 * z)

    return pl.pallas_call(
        body,
        out_shape=jax.ShapeDtypeStruct((_NP, 64), jnp.float32),
    )(acc2, y2, dinv, b2_row)


def _edge_layout(a):
    t0 = _NS * _KC0 * _CH
    t1 = _NS * _KC1 * _CH
    fill = jnp.full((t0 + t1 - a.shape[0],), _N, jnp.int32)
    flat = jnp.concatenate([a, fill])
    p0 = jnp.pad(flat[:t0].reshape(_NS, _KC0, _CH),
                 ((0, 0), (0, _KCM - _KC0), (0, 0)), constant_values=_N)
    p1 = jnp.pad(flat[t0:].reshape(_NS, _KC1, _CH),
                 ((0, 0), (0, _KCM - _KC1), (0, 0)), constant_values=_N)
    return jnp.concatenate([p0, p1], axis=0)


def kernel(x, edge_index, W1, b1, W2, b2):
    n = x.shape[0]
    e = edge_index.shape[1]
    assert e <= _NS * (_KC0 + _KC1) * _CH

    srcp = _edge_layout(edge_index[0])
    dstp = _edge_layout(edge_index[1])
    x_pad = jnp.pad(x, ((0, _NP - n), (0, 0)))
    mask = jax.random.bernoulli(jax.random.key(42), 0.5, (n, W1.shape[1]))
    scale = jnp.pad(jnp.where(mask, 2.0, 0.0).astype(jnp.float32),
                    ((0, _NP - n), (0, 0)))

    degp = _deg_kernel()(dstp)
    deg_col = 1.0 + jnp.sum(degp, axis=0)[:, None]
    y1, dinv = _tc_layer1(x_pad, deg_col, W1)
    acc1 = _scatter_kernel(32)(y1, srcp, dstp)
    y2 = _tc_layer2(acc1, y1, dinv, b1.reshape(1, -1), scale, W2)
    acc2 = _scatter_kernel(64)(y2, srcp, dstp)
    out = _tc_layer3(acc2, y2, dinv, b2.reshape(1, -1))
    return out[:n]


# speedup vs baseline: 1.5543x; 1.0013x over previous
import functools

import jax
import jax.numpy as jnp
from jax import lax
from jax.experimental import pallas as pl
from jax.experimental.pallas import tpu as pltpu
from jax.experimental.pallas import tpu_sc as plsc

_N = 10000
_NP = 10240
_NC = 2
_NS = 16
_NW = _NC * _NS
_CH = 128
_KC0 = 67
_KC1 = 90
_KCM = max(_KC0, _KC1)


def _sc_mesh():
    return plsc.VectorSubcoreMesh(
        core_axis_name="c", subcore_axis_name="s",
        num_cores=_NC, num_subcores=_NS)

_SC_PARAMS = dict(
    compiler_params=pltpu.CompilerParams(
        needs_layout_passes=False, use_tc_tiling_on_sc=False))


@functools.lru_cache(maxsize=None)
def _deg_kernel():
    @functools.partial(
        pl.kernel,
        out_type=jax.ShapeDtypeStruct((_NW, _NP), jnp.float32),
        mesh=_sc_mesh(),
        scratch_types=[
            pltpu.VMEM((_KCM, _CH), jnp.int32),
            pltpu.VMEM((_NP,), jnp.float32),
        ],
        **_SC_PARAMS,
    )
    def deg_k(dst_hbm, out_hbm, dst_v, deg_v):
        cid = lax.axis_index("c")
        sid = lax.axis_index("s")
        wid = cid * _NS + sid
        nb = jnp.where(cid == 0, _KC0, _KC1)
        pltpu.sync_copy(dst_hbm.at[wid], dst_v)
        zeros16 = jnp.zeros((16,), jnp.float32)

        @pl.loop(0, _NP // 16)
        def _(i):
            deg_v[pl.ds(i * 16, 16)] = zeros16

        ones16 = jnp.ones((16,), jnp.float32)

        @pl.loop(0, nb)
        def _(j):
            for c in range(_CH // 16):
                idx = dst_v[j, pl.ds(c * 16, 16)]
                plsc.addupdate_scatter(deg_v, [idx], ones16)

        pltpu.sync_copy(deg_v, out_hbm.at[wid])

    return deg_k


@functools.lru_cache(maxsize=None)
def _scatter_kernel(feat):
    rpt = _NP // _NS

    @functools.partial(
        pl.kernel,
        out_type=jax.ShapeDtypeStruct((_NC, _NP, feat), jnp.float32),
        mesh=_sc_mesh(),
        scratch_types=[
            pltpu.VMEM((_KCM, _CH), jnp.int32),
            pltpu.VMEM((_KCM, _CH), jnp.int32),
            pltpu.VMEM((_CH, feat), jnp.float32),
            pltpu.VMEM((rpt, feat), jnp.float32),
            pltpu.VMEM_SHARED((_NP, feat), jnp.float32),
            pltpu.SemaphoreType.DMA,
        ],
        **_SC_PARAMS,
    )
    def scat_k(tab_hbm, src_hbm, dst_hbm, out_hbm,
               src_v, dst_v, rows_v, zbuf, acc, gsem):
        cid = lax.axis_index("c")
        sid = lax.axis_index("s")
        wid = cid * _NS + sid
        nb = jnp.where(cid == 0, _KC0, _KC1)
        zeros16 = jnp.zeros((16,), jnp.float32)

        @pl.loop(0, rpt)
        def _(r):
            for c in range(feat // 16):
                zbuf[r, pl.ds(c * 16, 16)] = zeros16

        base = sid * rpt
        pltpu.sync_copy(zbuf, acc.at[pl.ds(base, rpt)])
        pltpu.sync_copy(src_hbm.at[wid], src_v)
        pltpu.sync_copy(dst_hbm.at[wid], dst_v)
        plsc.subcore_barrier()

        @pl.loop(0, nb)
        def _(j):
            pltpu.async_copy(tab_hbm.at[src_v.at[j]], rows_v, gsem).wait()
            pltpu.sync_copy(rows_v, acc.at[dst_v.at[j]], add=True)

        plsc.subcore_barrier()
        pltpu.sync_copy(acc.at[pl.ds(base, rpt)],
                        out_hbm.at[cid, pl.ds(base, rpt)])

    return scat_k


def _tc_mm1(x, w1):
    def body(x_ref, w_ref, o_ref):
        o_ref[0:_N, :] = jnp.dot(x_ref[...], w_ref[...],
                                 preferred_element_type=jnp.float32)
        o_ref[_N:_NP, :] = jnp.zeros((_NP - _N, w_ref.shape[1]),
                                     jnp.float32)

    return pl.pallas_call(
        body,
        out_shape=jax.ShapeDtypeStruct((_NP, 32), jnp.float32),
    )(x, w1)


def _tc_scale1(xw, deg_col):
    def body(xw_ref, d_ref, y_ref, dinv_ref):
        dinv = lax.rsqrt(d_ref[...])
        y_ref[...] = xw_ref[...] * dinv
        dinv_ref[...] = dinv

    return pl.pallas_call(
        body,
        out_shape=[
            jax.ShapeDtypeStruct((_NP, 32), jnp.float32),
            jax.ShapeDtypeStruct((_NP, 1), jnp.float32),
        ],
    )(xw, deg_col)


def _tc_layer2(acc1, y1, dinv, b1_row, scale, w2):
    def body(a_ref, y1_ref, dinv_ref, b_ref, s_ref, w_ref, y2_ref):
        dinv = dinv_ref[...]
        agg = a_ref[0] + a_ref[1] + y1_ref[...]
        h = agg * dinv + b_ref[...]
        h = jnp.maximum(h, 0.0) * s_ref[...]
        y2_ref[...] = jnp.dot(h, w_ref[...],
                              preferred_element_type=jnp.float32) * dinv

    return pl.pallas_call(
        body,
        out_shape=jax.ShapeDtypeStruct((_NP, 64), jnp.float32),
    )(acc1, y1, dinv, b1_row, scale, w2)


def _tc_layer3(acc2, y2, dinv, b2_row):
    def body(a_ref, y2_ref, dinv_ref, b_ref, o_ref):
        z = (a_ref[0] + a_ref[1] + y2_ref[...]) * dinv_ref[...] + b_ref[...]
        o_ref[...] = jnp.where(z > 0, z, 0.01 * z)

    return pl.pallas_call(
        body,
        out_shape=jax.ShapeDtypeStruct((_NP, 64), jnp.float32),
    )(acc2, y2, dinv, b2_row)


def _edge_layout(a):
    t0 = _NS * _KC0 * _CH
    t1 = _NS * _KC1 * _CH
    fill = jnp.full((t0 + t1 - a.shape[0],), _N, jnp.int32)
    flat = jnp.concatenate([a, fill])
    p0 = jnp.pad(flat[:t0].reshape(_NS, _KC0, _CH),
                 ((0, 0), (0, _KCM - _KC0), (0, 0)), constant_values=_N)
    p1 = jnp.pad(flat[t0:].reshape(_NS, _KC1, _CH),
                 ((0, 0), (0, _KCM - _KC1), (0, 0)), constant_values=_N)
    return jnp.concatenate([p0, p1], axis=0)


def kernel(x, edge_index, W1, b1, W2, b2):
    n = x.shape[0]
    e = edge_index.shape[1]
    assert e <= _NS * (_KC0 + _KC1) * _CH

    srcp = _edge_layout(edge_index[0])
    dstp = _edge_layout(edge_index[1])
    mask = jax.random.bernoulli(jax.random.key(42), 0.5, (n, W1.shape[1]))
    scale = jnp.pad(jnp.where(mask, 2.0, 0.0).astype(jnp.float32),
                    ((0, _NP - n), (0, 0)))

    xw1 = _tc_mm1(x, W1)
    degp = _deg_kernel()(dstp)
    deg_col = 1.0 + jnp.sum(degp, axis=0)[:, None]
    y1, dinv = _tc_scale1(xw1, deg_col)
    acc1 = _scatter_kernel(32)(y1, srcp, dstp)
    y2 = _tc_layer2(acc1, y1, dinv, b1.reshape(1, -1), scale, W2)
    acc2 = _scatter_kernel(64)(y2, srcp, dstp)
    out = _tc_layer3(acc2, y2, dinv, b2.reshape(1, -1))
    return out[:n]


# speedup vs baseline: 1.6848x; 1.0840x over previous
import functools

import jax
import jax.numpy as jnp
from jax import lax
from jax.experimental import pallas as pl
from jax.experimental.pallas import tpu as pltpu
from jax.experimental.pallas import tpu_sc as plsc

_N = 10000
_NP = 10240
_NC = 2
_NS = 16
_NW = _NC * _NS
_CH = 128
_KC0 = 90
_KC1 = 67
_KCM = max(_KC0, _KC1)


def _sc_mesh():
    return plsc.VectorSubcoreMesh(
        core_axis_name="c", subcore_axis_name="s",
        num_cores=_NC, num_subcores=_NS)

_SC_PARAMS = dict(
    compiler_params=pltpu.CompilerParams(
        needs_layout_passes=False, use_tc_tiling_on_sc=False))


@functools.lru_cache(maxsize=None)
def _deg_kernel():
    @functools.partial(
        pl.kernel,
        out_type=jax.ShapeDtypeStruct((_NW, _NP), jnp.float32),
        mesh=_sc_mesh(),
        scratch_types=[
            pltpu.VMEM((_KCM, _CH), jnp.int32),
            pltpu.VMEM((_NP,), jnp.float32),
        ],
        **_SC_PARAMS,
    )
    def deg_k(dst_hbm, out_hbm, dst_v, deg_v):
        cid = lax.axis_index("c")
        sid = lax.axis_index("s")
        wid = cid * _NS + sid
        nb = jnp.where(cid == 0, _KC0, _KC1)
        pltpu.sync_copy(dst_hbm.at[wid], dst_v)
        zeros16 = jnp.zeros((16,), jnp.float32)

        @pl.loop(0, _NP // 16)
        def _(i):
            deg_v[pl.ds(i * 16, 16)] = zeros16

        ones16 = jnp.ones((16,), jnp.float32)

        @pl.loop(0, nb)
        def _(j):
            for c in range(_CH // 16):
                idx = dst_v[j, pl.ds(c * 16, 16)]
                plsc.addupdate_scatter(deg_v, [idx], ones16)

        pltpu.sync_copy(deg_v, out_hbm.at[wid])

    return deg_k


@functools.lru_cache(maxsize=None)
def _scatter_kernel(feat):
    rpt = _NP // _NS

    @functools.partial(
        pl.kernel,
        out_type=jax.ShapeDtypeStruct((_NC, _NP, feat), jnp.float32),
        mesh=_sc_mesh(),
        scratch_types=[
            pltpu.VMEM((_KCM, _CH), jnp.int32),
            pltpu.VMEM((_KCM, _CH), jnp.int32),
            pltpu.VMEM((_CH, feat), jnp.float32),
            pltpu.VMEM((rpt, feat), jnp.float32),
            pltpu.VMEM_SHARED((_NP, feat), jnp.float32),
            pltpu.SemaphoreType.DMA,
        ],
        **_SC_PARAMS,
    )
    def scat_k(tab_hbm, src_hbm, dst_hbm, out_hbm,
               src_v, dst_v, rows_v, zbuf, acc, gsem):
        cid = lax.axis_index("c")
        sid = lax.axis_index("s")
        wid = cid * _NS + sid
        nb = jnp.where(cid == 0, _KC0, _KC1)
        zeros16 = jnp.zeros((16,), jnp.float32)

        @pl.loop(0, rpt)
        def _(r):
            for c in range(feat // 16):
                zbuf[r, pl.ds(c * 16, 16)] = zeros16

        base = sid * rpt
        pltpu.sync_copy(zbuf, acc.at[pl.ds(base, rpt)])
        pltpu.sync_copy(src_hbm.at[wid], src_v)
        pltpu.sync_copy(dst_hbm.at[wid], dst_v)
        plsc.subcore_barrier()

        @pl.loop(0, nb)
        def _(j):
            pltpu.async_copy(tab_hbm.at[src_v.at[j]], rows_v, gsem).wait()
            pltpu.sync_copy(rows_v, acc.at[dst_v.at[j]], add=True)

        plsc.subcore_barrier()
        pltpu.sync_copy(acc.at[pl.ds(base, rpt)],
                        out_hbm.at[cid, pl.ds(base, rpt)])

    return scat_k


def _tc_mm1(x, w1):
    def body(x_ref, w_ref, o_ref):
        o_ref[0:_N, :] = jnp.dot(x_ref[...], w_ref[...],
                                 preferred_element_type=jnp.float32)
        o_ref[_N:_NP, :] = jnp.zeros((_NP - _N, w_ref.shape[1]),
                                     jnp.float32)

    return pl.pallas_call(
        body,
        out_shape=jax.ShapeDtypeStruct((_NP, 32), jnp.float32),
    )(x, w1)


def _tc_scale1(xw, deg_col):
    def body(xw_ref, d_ref, y_ref, dinv_ref):
        dinv = lax.rsqrt(d_ref[...])
        y_ref[...] = xw_ref[...] * dinv
        dinv_ref[...] = dinv

    return pl.pallas_call(
        body,
        out_shape=[
            jax.ShapeDtypeStruct((_NP, 32), jnp.float32),
            jax.ShapeDtypeStruct((_NP, 1), jnp.float32),
        ],
    )(xw, deg_col)


def _tc_layer2(acc1, y1, dinv, b1_row, scale, w2):
    def body(a_ref, y1_ref, dinv_ref, b_ref, s_ref, w_ref, y2_ref):
        dinv = dinv_ref[...]
        agg = a_ref[0] + a_ref[1] + y1_ref[...]
        h = agg * dinv + b_ref[...]
        h = jnp.maximum(h, 0.0) * s_ref[...]
        y2_ref[...] = jnp.dot(h, w_ref[...],
                              preferred_element_type=jnp.float32) * dinv

    return pl.pallas_call(
        body,
        out_shape=jax.ShapeDtypeStruct((_NP, 64), jnp.float32),
    )(acc1, y1, dinv, b1_row, scale, w2)


def _tc_layer3(acc2, y2, dinv, b2_row):
    def body(a_ref, y2_ref, dinv_ref, b_ref, o_ref):
        z = (a_ref[0] + a_ref[1] + y2_ref[...]) * dinv_ref[...] + b_ref[...]
        o_ref[...] = jnp.where(z > 0, z, 0.01 * z)

    return pl.pallas_call(
        body,
        out_shape=jax.ShapeDtypeStruct((_NP, 64), jnp.float32),
    )(acc2, y2, dinv, b2_row)


def _edge_layout(a):
    t0 = _NS * _KC0 * _CH
    t1 = _NS * _KC1 * _CH
    fill = jnp.full((t0 + t1 - a.shape[0],), _N, jnp.int32)
    flat = jnp.concatenate([a, fill])
    p0 = jnp.pad(flat[:t0].reshape(_NS, _KC0, _CH),
                 ((0, 0), (0, _KCM - _KC0), (0, 0)), constant_values=_N)
    p1 = jnp.pad(flat[t0:].reshape(_NS, _KC1, _CH),
                 ((0, 0), (0, _KCM - _KC1), (0, 0)), constant_values=_N)
    return jnp.concatenate([p0, p1], axis=0)


def kernel(x, edge_index, W1, b1, W2, b2):
    n = x.shape[0]
    e = edge_index.shape[1]
    assert e <= _NS * (_KC0 + _KC1) * _CH

    srcp = _edge_layout(edge_index[0])
    dstp = _edge_layout(edge_index[1])
    mask = jax.random.bernoulli(jax.random.key(42), 0.5, (n, W1.shape[1]))
    scale = jnp.pad(jnp.where(mask, 2.0, 0.0).astype(jnp.float32),
                    ((0, _NP - n), (0, 0)))

    xw1 = _tc_mm1(x, W1)
    degp = _deg_kernel()(dstp)
    deg_col = 1.0 + jnp.sum(degp, axis=0)[:, None]
    y1, dinv = _tc_scale1(xw1, deg_col)
    acc1 = _scatter_kernel(32)(y1, srcp, dstp)
    y2 = _tc_layer2(acc1, y1, dinv, b1.reshape(1, -1), scale, W2)
    acc2 = _scatter_kernel(64)(y2, srcp, dstp)
    out = _tc_layer3(acc2, y2, dinv, b2.reshape(1, -1))
    return out[:n]
